# Initial kernel scaffold; baseline (speedup 1.0000x reference)
import jax
import jax.numpy as jnp
from jax.experimental import pallas as pl


def kernel(prev_labels, mask, logits):
    raise NotImplementedError("write your pallas kernel here")


# speedup vs baseline: 1.8076x; 1.8076x over previous
import functools

import jax
import jax.numpy as jnp
from jax import lax
from jax.experimental import pallas as pl
from jax.experimental.pallas import tpu as pltpu
from jax.experimental.pallas import tpu_sc as plsc

_K = 32
_B = 4096 * 200
_NW = 32
_CHUNK = 128
_NCHUNK = _B // (_NW * _CHUNK)


def _table_body(mask_ref, logits_ref, out_ref):
    masked = jnp.where(mask_ref[...] == 0.0, jnp.float32(-50.0), logits_ref[...])
    m = jnp.max(masked, axis=1, keepdims=True)
    s = masked - m
    out_ref[...] = s - jnp.log(jnp.sum(jnp.exp(s), axis=1, keepdims=True))


def _compute_table(mask, logits):
    return pl.pallas_call(
        _table_body,
        out_shape=jax.ShapeDtypeStruct((_K, _K), jnp.float32),
    )(mask, logits)


def _sc_gather(table, idx2d):
    mesh = plsc.VectorSubcoreMesh(core_axis_name="c", subcore_axis_name="s")

    @functools.partial(
        pl.kernel,
        mesh=mesh,
        out_type=jax.ShapeDtypeStruct((_B, _K), jnp.float32),
        scratch_types=[
            pltpu.VMEM((_NCHUNK, _CHUNK), jnp.int32),
            pltpu.VMEM((_CHUNK, _K), jnp.float32),
            pltpu.SemaphoreType.DMA,
        ],
        compiler_params=pltpu.CompilerParams(use_tc_tiling_on_sc=False),
    )
    def k(table_hbm, idx_hbm, out_hbm, idx_v, rows_v, sem):
        wid = lax.axis_index("s") * 2 + lax.axis_index("c")
        row0 = wid * _NCHUNK
        pltpu.sync_copy(idx_hbm.at[pl.ds(row0, _NCHUNK)], idx_v)

        def body(j, carry):
            pltpu.async_copy(table_hbm.at[idx_v.at[j]], rows_v, sem).wait()
            pltpu.sync_copy(rows_v, out_hbm.at[pl.ds((row0 + j) * _CHUNK, _CHUNK)])
            return carry

        lax.fori_loop(0, _NCHUNK, body, 0)

    return k(table, idx2d)


def kernel(prev_labels, mask, logits):
    table = _compute_table(mask.astype(jnp.float32), logits.astype(jnp.float32))
    idx2d = prev_labels.astype(jnp.int32).reshape(_NW * _NCHUNK, _CHUNK)
    out = _sc_gather(table, idx2d)
    return out.reshape(prev_labels.shape + (_K,))


# speedup vs baseline: 1.8353x; 1.0153x over previous
import functools

import jax
import jax.numpy as jnp
from jax import lax
from jax.experimental import pallas as pl
from jax.experimental.pallas import tpu as pltpu
from jax.experimental.pallas import tpu_sc as plsc

_K = 32
_B = 4096 * 200
_NW = 32
_CHUNK = 128
_NCHUNK = _B // (_NW * _CHUNK)


def _table_body(mask_ref, logits_ref, out_ref):
    masked = jnp.where(mask_ref[...] == 0.0, jnp.float32(-50.0), logits_ref[...])
    m = jnp.max(masked, axis=1, keepdims=True)
    s = masked - m
    out_ref[...] = s - jnp.log(jnp.sum(jnp.exp(s), axis=1, keepdims=True))


def _compute_table(mask, logits):
    return pl.pallas_call(
        _table_body,
        out_shape=jax.ShapeDtypeStruct((_K, _K), jnp.float32),
    )(mask, logits)


_KFIRE = 10
_GROUPS = _NCHUNK // _KFIRE
_PAIRS = _GROUPS // 2
_GROWS = _KFIRE * _CHUNK


def _sc_gather(table, idx2d):
    mesh = plsc.VectorSubcoreMesh(core_axis_name="c", subcore_axis_name="s")

    @functools.partial(
        pl.kernel,
        mesh=mesh,
        out_type=jax.ShapeDtypeStruct((_B, _K), jnp.float32),
        scratch_types=[
            pltpu.VMEM((_NCHUNK, _CHUNK), jnp.int32),
            pltpu.VMEM((2, _GROWS, _K), jnp.float32),
            pltpu.SemaphoreType.DMA,
            pltpu.SemaphoreType.DMA,
            pltpu.SemaphoreType.DMA,
            pltpu.SemaphoreType.DMA,
        ],
        compiler_params=pltpu.CompilerParams(use_tc_tiling_on_sc=False),
    )
    def k(table_hbm, idx_hbm, out_hbm, idx_v, rows_v, g0, g1, o0, o1, ):
        gsem = (g0, g1)
        osem = (o0, o1)
        wid = lax.axis_index("s") * 2 + lax.axis_index("c")
        row0 = wid * _NCHUNK
        pltpu.sync_copy(idx_hbm.at[pl.ds(row0, _NCHUNK)], idx_v)

        def out_slice(g):
            return out_hbm.at[pl.ds((row0 + g * _KFIRE) * _CHUNK, _GROWS)]

        def fire(g, b):
            for j in range(_KFIRE):
                pltpu.async_copy(
                    table_hbm.at[idx_v.at[g * _KFIRE + j]],
                    rows_v.at[b, pl.ds(j * _CHUNK, _CHUNK)],
                    gsem[b],
                )

        def wait_gathers(g, b):
            pltpu.make_async_copy(out_slice(g), rows_v.at[b], gsem[b]).wait()

        def start_out(g, b):
            pltpu.async_copy(rows_v.at[b], out_slice(g), osem[b])

        def wait_out(g, b):
            pltpu.make_async_copy(rows_v.at[b], out_slice(g), osem[b]).wait()

        def pair(p, carry):
            for b in (0, 1):
                g = 2 * p + b

                @pl.when(p > 0)
                def _():
                    wait_out(g, b)

                fire(g, b)
            for b in (0, 1):
                g = 2 * p + b
                wait_gathers(g, b)
                start_out(g, b)
            return carry

        lax.fori_loop(0, _PAIRS, pair, 0)
        last = 2 * _PAIRS - 2
        wait_out(last, 0)
        wait_out(last + 1, 1)

    return k(table, idx2d)


def kernel(prev_labels, mask, logits):
    table = _compute_table(mask.astype(jnp.float32), logits.astype(jnp.float32))
    idx2d = prev_labels.astype(jnp.int32).reshape(_NW * _NCHUNK, _CHUNK)
    out = _sc_gather(table, idx2d)
    return out.reshape(prev_labels.shape + (_K,))


# speedup vs baseline: 2.1009x; 1.1447x over previous
import functools

import jax
import jax.numpy as jnp
from jax import lax
from jax.experimental import pallas as pl
from jax.experimental.pallas import tpu as pltpu
from jax.experimental.pallas import tpu_sc as plsc

_K = 32
_B = 4096 * 200
_NW = 32
_PERW = _B // _NW
_GROWS = 1280
_GROUPS = _PERW // _GROWS
_PAIRS = _GROUPS // 2
_BLKS = _GROWS // 16


def _table_body(mask_ref, logits_ref, out_ref):
    masked = jnp.where(mask_ref[...] == 0.0, jnp.float32(-50.0), logits_ref[...])
    m = jnp.max(masked, axis=1, keepdims=True)
    s = masked - m
    out_ref[...] = s - jnp.log(jnp.sum(jnp.exp(s), axis=1, keepdims=True))


def _compute_table(mask, logits):
    return pl.pallas_call(
        _table_body,
        out_shape=jax.ShapeDtypeStruct((_K, _K), jnp.float32),
    )(mask, logits)


def _sc_expand(table_flat, idx_flat):
    mesh = plsc.VectorSubcoreMesh(core_axis_name="c", subcore_axis_name="s")

    @functools.partial(
        pl.kernel,
        mesh=mesh,
        out_type=jax.ShapeDtypeStruct((_B * _K,), jnp.float32),
        scratch_types=[
            pltpu.VMEM((_K * _K,), jnp.float32),
            pltpu.VMEM((_PERW,), jnp.int32),
            pltpu.VMEM((2, _GROWS * _K), jnp.float32),
            pltpu.SemaphoreType.DMA,
            pltpu.SemaphoreType.DMA,
        ],
        compiler_params=pltpu.CompilerParams(
            use_tc_tiling_on_sc=False, needs_layout_passes=False
        ),
    )
    def k(table_hbm, idx_hbm, out_hbm, table_v, idx_v, out_v, o0, o1):
        osem = (o0, o1)
        wid = lax.axis_index("s") * 2 + lax.axis_index("c")
        idx0 = wid * _PERW
        pltpu.sync_copy(idx_hbm.at[pl.ds(idx0, _PERW)], idx_v)
        pltpu.sync_copy(table_hbm, table_v)
        iota = lax.iota(jnp.int32, 16)

        def out_slice(g):
            return out_hbm.at[pl.ds((idx0 + g * _GROWS) * _K, _GROWS * _K)]

        def build(g, b):
            def blk_body(blk, carry):
                idxv = idx_v[pl.ds(g * _GROWS + blk * 16, 16)]
                a32 = idxv * _K
                rowv = iota * _K + blk * (16 * _K)
                for c in range(_K):
                    vals = plsc.load_gather(table_v, [a32 + c])
                    plsc.store_scatter(out_v.at[b], [rowv + c], vals)
                return carry

            lax.fori_loop(0, _BLKS, blk_body, 0)

        def start_out(g, b):
            pltpu.async_copy(out_v.at[b], out_slice(g), osem[b])

        def wait_out(g, b):
            pltpu.make_async_copy(out_v.at[b], out_slice(g), osem[b]).wait()

        def pair(p, carry):
            for b in (0, 1):
                g = 2 * p + b

                @pl.when(p > 0)
                def _():
                    wait_out(g, b)

                build(g, b)
                start_out(g, b)
            return carry

        lax.fori_loop(0, _PAIRS, pair, 0)
        last = 2 * _PAIRS - 2
        wait_out(last, 0)
        wait_out(last + 1, 1)

    return k(table_flat, idx_flat)


def kernel(prev_labels, mask, logits):
    table = _compute_table(mask.astype(jnp.float32), logits.astype(jnp.float32))
    idx_flat = prev_labels.astype(jnp.int32).reshape(_B)
    out = _sc_expand(table.reshape(_K * _K), idx_flat)
    return out.reshape(prev_labels.shape + (_K,))


# speedup vs baseline: 4.5405x; 2.1612x over previous
import functools

import jax
import jax.numpy as jnp
from jax import lax
from jax.experimental import pallas as pl
from jax.experimental.pallas import tpu as pltpu
from jax.experimental.pallas import tpu_sc as plsc

_K = 32
_B = 4096 * 200
_NW = 32
_PERW = _B // _NW
_GROWS = 1280
_GROUPS = _PERW // _GROWS
_PAIRS = _GROUPS // 2
_BLKS = _GROWS // 16


def _table_body(mask_ref, logits_ref, out_ref):
    masked = jnp.where(mask_ref[...] == 0.0, jnp.float32(-50.0), logits_ref[...])
    m = jnp.max(masked, axis=1, keepdims=True)
    s = masked - m
    out_ref[...] = s - jnp.log(jnp.sum(jnp.exp(s), axis=1, keepdims=True))


def _compute_table(mask, logits):
    return pl.pallas_call(
        _table_body,
        out_shape=jax.ShapeDtypeStruct((_K, _K), jnp.float32),
    )(mask, logits)


def _sc_expand(table_flat, idx_flat):
    mesh = plsc.VectorSubcoreMesh(core_axis_name="c", subcore_axis_name="s")

    @functools.partial(
        pl.kernel,
        mesh=mesh,
        out_type=jax.ShapeDtypeStruct((_B * _K,), jnp.float32),
        scratch_types=[
            pltpu.VMEM((_K * _K,), jnp.float32),
            pltpu.VMEM((_PERW,), jnp.int32),
            pltpu.VMEM((2, _GROWS * _K), jnp.float32),
            pltpu.SemaphoreType.DMA,
            pltpu.SemaphoreType.DMA,
        ],
        compiler_params=pltpu.CompilerParams(
            use_tc_tiling_on_sc=False, needs_layout_passes=False
        ),
    )
    def k(table_hbm, idx_hbm, out_hbm, table_v, idx_v, out_v, o0, o1):
        osem = (o0, o1)
        wid = lax.axis_index("s") * 2 + lax.axis_index("c")
        idx0 = wid * _PERW
        pltpu.sync_copy(idx_hbm.at[pl.ds(idx0, _PERW)], idx_v)
        pltpu.sync_copy(table_hbm, table_v)

        def out_slice(g):
            return out_hbm.at[pl.ds((idx0 + g * _GROWS) * _K, _GROWS * _K)]

        def build(g, b):
            def blk_body(blk, carry):
                base = g * _GROWS + blk * 16
                obase = blk * (16 * _K)
                idxv = idx_v[pl.ds(base, 16)] * _K
                for r in range(16):
                    t = idxv[r]
                    o = obase + r * _K
                    out_v[b, pl.ds(o, 16)] = table_v[pl.ds(t, 16)]
                    out_v[b, pl.ds(o + 16, 16)] = table_v[pl.ds(t + 16, 16)]
                return carry

            lax.fori_loop(0, _BLKS, blk_body, 0)

        def start_out(g, b):
            pltpu.async_copy(out_v.at[b], out_slice(g), osem[b])

        def wait_out(g, b):
            pltpu.make_async_copy(out_v.at[b], out_slice(g), osem[b]).wait()

        def pair(p, carry):
            for b in (0, 1):
                g = 2 * p + b

                @pl.when(p > 0)
                def _():
                    wait_out(g, b)

                build(g, b)
                start_out(g, b)
            return carry

        lax.fori_loop(0, _PAIRS, pair, 0)
        last = 2 * _PAIRS - 2
        wait_out(last, 0)
        wait_out(last + 1, 1)

    return k(table_flat, idx_flat)


def kernel(prev_labels, mask, logits):
    table = _compute_table(mask.astype(jnp.float32), logits.astype(jnp.float32))
    idx_flat = prev_labels.astype(jnp.int32).reshape(_B)
    out = _sc_expand(table.reshape(_K * _K), idx_flat)
    return out.reshape(prev_labels.shape + (_K,))


# speedup vs baseline: 4.6850x; 1.0318x over previous
import functools

import jax
import jax.numpy as jnp
from jax import lax
from jax.experimental import pallas as pl
from jax.experimental.pallas import tpu as pltpu
from jax.experimental.pallas import tpu_sc as plsc

_K = 32
_B = 4096 * 200
_NW = 32
_PERW = _B // _NW
_CHUNK = 128
_NCHUNK = _PERW // _CHUNK
_KFIRE = 10
_GROUPS = _NCHUNK // _KFIRE
_PAIRS = _GROUPS // 2
_GROWS = _KFIRE * _CHUNK


def _table_body(mask_ref, logits_ref, out_ref):
    masked = jnp.where(mask_ref[...] == 0.0, jnp.float32(-50.0), logits_ref[...])
    m = jnp.max(masked, axis=1, keepdims=True)
    s = masked - m
    lp = s - jnp.log(jnp.sum(jnp.exp(s), axis=1, keepdims=True))
    out_ref[...] = jnp.broadcast_to(lp[None], (_NW, _K, _K))


def _compute_table_replicated(mask, logits):
    return pl.pallas_call(
        _table_body,
        out_shape=jax.ShapeDtypeStruct((_NW, _K, _K), jnp.float32),
    )(mask, logits)


def _sc_gather(table_rep, idx2d):
    mesh = plsc.VectorSubcoreMesh(core_axis_name="c", subcore_axis_name="s")

    @functools.partial(
        pl.kernel,
        mesh=mesh,
        out_type=jax.ShapeDtypeStruct((_B, _K), jnp.float32),
        scratch_types=[
            pltpu.VMEM((_NCHUNK, _CHUNK), jnp.int32),
            pltpu.VMEM((2, _GROWS, _K), jnp.float32),
            pltpu.SemaphoreType.DMA,
            pltpu.SemaphoreType.DMA,
            pltpu.SemaphoreType.DMA,
            pltpu.SemaphoreType.DMA,
        ],
        compiler_params=pltpu.CompilerParams(
            use_tc_tiling_on_sc=False, needs_layout_passes=False
        ),
    )
    def k(table_hbm, idx_hbm, out_hbm, idx_v, rows_v, g0, g1, o0, o1):
        gsem = (g0, g1)
        osem = (o0, o1)
        wid = lax.axis_index("s") * 2 + lax.axis_index("c")
        row0 = wid * _NCHUNK
        pltpu.sync_copy(idx_hbm.at[pl.ds(row0, _NCHUNK)], idx_v)

        woff = wid * _K

        def adjust(j, carry):
            for q in range(_CHUNK // 16):
                sl = pl.ds(q * 16, 16)
                idx_v[j, sl] = idx_v[j, sl] + woff
            return carry

        lax.fori_loop(0, _NCHUNK, adjust, 0)

        def out_slice(g):
            return out_hbm.at[pl.ds((row0 + g * _KFIRE) * _CHUNK, _GROWS)]

        def fire(g, b):
            for j in range(_KFIRE):
                pltpu.async_copy(
                    table_hbm.at[idx_v.at[g * _KFIRE + j]],
                    rows_v.at[b, pl.ds(j * _CHUNK, _CHUNK)],
                    gsem[b],
                )

        def wait_gathers(g, b):
            pltpu.make_async_copy(out_slice(g), rows_v.at[b], gsem[b]).wait()

        def start_out(g, b):
            pltpu.async_copy(rows_v.at[b], out_slice(g), osem[b])

        def wait_out(g, b):
            pltpu.make_async_copy(rows_v.at[b], out_slice(g), osem[b]).wait()

        def pair(p, carry):
            for b in (0, 1):
                g = 2 * p + b

                @pl.when(p > 0)
                def _():
                    wait_out(g, b)

                fire(g, b)
            for b in (0, 1):
                g = 2 * p + b
                wait_gathers(g, b)
                start_out(g, b)
            return carry

        lax.fori_loop(0, _PAIRS, pair, 0)
        last = 2 * _PAIRS - 2
        wait_out(last, 0)
        wait_out(last + 1, 1)

    return k(table_rep, idx2d)


def kernel(prev_labels, mask, logits):
    table_rep = _compute_table_replicated(
        mask.astype(jnp.float32), logits.astype(jnp.float32)
    ).reshape(_NW * _K, _K)
    idx2d = prev_labels.astype(jnp.int32).reshape(_NW * _NCHUNK, _CHUNK)
    out = _sc_gather(table_rep, idx2d)
    return out.reshape(prev_labels.shape + (_K,))
